# baseline (speedup 1.0000x reference)
import functools

import jax
import jax.numpy as jnp
from jax import lax
from jax.experimental import pallas as pl
from jax.experimental.pallas import tpu as pltpu
from jax.experimental.pallas import tpu_sc as plsc

NUM_SUB = 8
K = 8192
SUB_DIM = 64
DIM = NUM_SUB * SUB_DIM
NUM_CODES = 100000
R = 40
FLAT = R * NUM_SUB
NUM_CHUNKS = NUM_CODES // R
LANES = 16
TILE_ROWS = R // 8
LANE_BLKS = DIM // 128


def _make_gather_kernel():
    info = plsc.get_sparse_core_info()
    nc, ns = info.num_cores, info.num_subcores
    nw = nc * ns
    max_mine = -(-NUM_CHUNKS // nw)
    n_pairs = -(-max_mine // 2)
    mesh = plsc.VectorSubcoreMesh(core_axis_name="c", subcore_axis_name="s")

    @functools.partial(
        pl.kernel,
        out_type=jax.ShapeDtypeStruct((NUM_CODES // 8, LANE_BLKS, 8, 128),
                                      jnp.float32),
        mesh=mesh,
        scratch_types=[
            pltpu.VMEM((FLAT,), jnp.int32),
            pltpu.VMEM((FLAT,), jnp.int32),
            pltpu.VMEM((FLAT, SUB_DIM), jnp.float32),
            pltpu.VMEM((FLAT, SUB_DIM), jnp.float32),
            pltpu.VMEM((TILE_ROWS, LANE_BLKS, 8, 128), jnp.float32),
            pltpu.VMEM((TILE_ROWS, LANE_BLKS, 8, 128), jnp.float32),
            pltpu.SemaphoreType.DMA,
            pltpu.SemaphoreType.DMA,
            pltpu.SemaphoreType.DMA,
            pltpu.SemaphoreType.DMA,
        ],
        compiler_params=pltpu.CompilerParams(use_tc_tiling_on_sc=False),
    )
    def gather_kernel(table_hbm, code_hbm, out_hbm,
                      idx0, idx1, gb0, gb1, tb0, tb1, g0, g1, w0, w1):
        wid = lax.axis_index("s") * nc + lax.axis_index("c")
        idx_b, gbuf_b, tbuf_b = (idx0, idx1), (gb0, gb1), (tb0, tb1)
        gsem, wsem = (g0, g1), (w0, w1)
        offs = (lax.broadcasted_iota(jnp.int32, (LANES,), 0) & 7) * K
        n_mine = (NUM_CHUNKS - wid + nw - 1) // nw

        def load(t, b):
            pltpu.sync_copy(code_hbm.at[wid + t * nw], idx_b[b])

            def add_offs(g, c):
                sl = pl.ds(g * LANES, LANES)
                idx_b[b][sl] = idx_b[b][sl] + offs
                return c

            lax.fori_loop(0, FLAT // LANES, add_offs, 0, unroll=True)

            @pl.when(t >= 2)
            def _():
                pltpu.make_async_copy(
                    tbuf_b[b], out_hbm.at[pl.ds(0, TILE_ROWS)],
                    wsem[b]).wait()

            pltpu.async_copy(table_hbm.at[idx_b[b]], gbuf_b[b], gsem[b])

        def store(t, b):
            pltpu.make_async_copy(
                table_hbm.at[idx_b[b]], gbuf_b[b], gsem[b]).wait()

            def rearrange(n, c):
                u, r = n >> 3, n & 7
                base = n * NUM_SUB
                for j in range(LANE_BLKS):
                    for h in range(2):
                        for w in range(SUB_DIM // LANES):
                            v = gbuf_b[b][base + 2 * j + h,
                                          pl.ds(w * LANES, LANES)]
                            tbuf_b[b][u, j, r,
                                      pl.ds(h * SUB_DIM + w * LANES,
                                            LANES)] = v
                return c

            lax.fori_loop(0, R, rearrange, 0)
            tr0 = (wid + t * nw) * TILE_ROWS
            pltpu.async_copy(
                tbuf_b[b], out_hbm.at[pl.ds(tr0, TILE_ROWS)], wsem[b])

        load(0, 0)

        def pair(g, carry):
            t0, t1 = 2 * g, 2 * g + 1

            @pl.when(t1 < n_mine)
            def _():
                load(t1, 1)

            @pl.when(t0 < n_mine)
            def _():
                store(t0, 0)

            @pl.when(t1 + 1 < n_mine)
            def _():
                load(t1 + 1, 0)

            @pl.when(t1 < n_mine)
            def _():
                store(t1, 1)

            return carry

        lax.fori_loop(0, n_pairs, pair, 0)
        for b in (0, 1):
            pltpu.make_async_copy(
                tbuf_b[b], out_hbm.at[pl.ds(0, TILE_ROWS)], wsem[b]).wait()

    return gather_kernel


_gather = _make_gather_kernel()


@jax.jit
def kernel(code, centroid):
    table = centroid.reshape(NUM_SUB * K, SUB_DIM)
    code2 = code.astype(jnp.int32).reshape(NUM_CHUNKS, FLAT)
    tiled = _gather(table, code2)
    return tiled.transpose(0, 2, 1, 3).reshape(NUM_CODES, DIM)
